# baseline (speedup 1.0000x reference)
import functools

import jax
import jax.numpy as jnp
from jax import lax
from jax.experimental import pallas as pl
from jax.experimental.pallas import tpu as pltpu
from jax.experimental.pallas import tpu_sc as plsc

BATCH = 4096
SEQ = 50
EMB = 64
TOTAL = BATCH * SEQ
NC = 2
NS = 16
NW = NC * NS
BPW = TOTAL // NW
CHUNK = 128
NCHUNK = BPW // CHUNK
GROUP = 5
NGROUP = NCHUNK // GROUP
GROUP_ROWS = GROUP * CHUNK


def _gather_body(table_hbm, idx_hbm, out_hbm, idx_v, rows_v, sem):
    wid = lax.axis_index("s") * NC + lax.axis_index("c")
    base_row = wid * BPW

    pltpu.sync_copy(idx_hbm.at[wid], idx_v)

    def group(g, carry):
        copies = []
        for j in range(GROUP):
            copies.append(pltpu.async_copy(
                table_hbm.at[idx_v.at[g * GROUP + j]],
                rows_v.at[pl.ds(j * CHUNK, CHUNK)],
                sem))
        for c in copies:
            c.wait()
        pltpu.sync_copy(rows_v,
                        out_hbm.at[pl.ds(base_row + g * GROUP_ROWS, GROUP_ROWS)])
        return carry

    lax.fori_loop(0, NGROUP, group, 0)


@jax.jit
def _run(table, idx2d):
    mesh = plsc.VectorSubcoreMesh(core_axis_name="c", subcore_axis_name="s")
    fn = pl.kernel(
        _gather_body,
        mesh=mesh,
        out_type=jax.ShapeDtypeStruct((TOTAL, EMB), jnp.float32),
        scratch_types=[
            pltpu.VMEM((NCHUNK, CHUNK), jnp.int32),
            pltpu.VMEM((GROUP_ROWS, EMB), jnp.float32),
            pltpu.SemaphoreType.DMA,
        ],
        compiler_params=pltpu.CompilerParams(use_tc_tiling_on_sc=False),
    )
    return fn(table, idx2d)


def kernel(image_features, captions, embedding_weight):
    idx3d = captions.astype(jnp.int32).reshape(NW, NCHUNK, CHUNK)
    out = _run(embedding_weight, idx3d)
    return out.reshape(BATCH, SEQ, EMB)
